# baseline (speedup 1.0000x reference)
import jax
import jax.numpy as jnp
from jax.experimental import pallas as pl
from jax.experimental.pallas import tpu as pltpu

_R = 16
_NEG_INF = float("-inf")


def _topk_onehot(scores, k):
    bsz, n_exp = scores.shape
    col = jax.lax.broadcasted_iota(jnp.int32, (bsz, n_exp), 1)
    run = scores
    outs = []
    for _ in range(k):
        m = jnp.max(run, axis=1, keepdims=True)
        cand = jnp.where(run == m, col, n_exp)
        amin = jnp.min(cand, axis=1, keepdims=True)
        oh = col == amin
        outs.append(oh.astype(scores.dtype))
        run = jnp.where(oh, _NEG_INF, run)
    return outs


def _routing_kernel(q_ref, wa_ref, ba_ref, wb_ref, bb_ref, apool_ref,
                    cfs_ref, bt_ref, lora_a_ref, b_sel_ref):
    q = q_ref[...]
    s_a = jax.lax.dot_general(q, wa_ref[...], (((1,), (1,)), ((), ())),
                              preferred_element_type=jnp.float32)
    s_a = s_a + ba_ref[...]
    oh_a = _topk_onehot(s_a, _R)

    bsz = q.shape[0]
    n_exp = wa_ref.shape[0]
    g_cfs = jnp.zeros((bsz, n_exp), dtype=jnp.float32)
    for r in range(_R):
        la_r = jnp.dot(oh_a[r], apool_ref[:, r, :],
                       preferred_element_type=jnp.float32)
        lora_a_ref[:, r, :] = la_r
        g_cfs = g_cfs + jnp.dot(la_r, cfs_ref[r],
                                preferred_element_type=jnp.float32)

    s_b = jax.lax.dot_general(q, wb_ref[...], (((1,), (1,)), ((), ())),
                              preferred_element_type=jnp.float32)
    s_b = s_b + bb_ref[...] + g_cfs
    oh_b = _topk_onehot(s_b, _R)
    for r in range(_R):
        b_sel_ref[:, r, :] = jnp.dot(oh_b[r], bt_ref[r],
                                     preferred_element_type=jnp.float32)


def _apply_kernel(x_ref, la_ref, bs_ref, out_ref):
    x = x_ref[0]
    la = la_ref[0]
    bs = bs_ref[0]
    after = jax.lax.dot_general(x, la, (((1,), (1,)), ((), ())),
                                preferred_element_type=jnp.float32)
    out_ref[0] = jnp.dot(after, bs, preferred_element_type=jnp.float32)


def _build(interpret=False):
    def run(x, query_signal, A_pool, B_pool, W_A, b_A, W_B, b_B, cfs_W):
        bsz, seq, d_in = x.shape
        n_exp = A_pool.shape[0]
        d_out = B_pool.shape[1]
        bt = jnp.transpose(B_pool, (2, 0, 1))
        lora_a, b_sel = pl.pallas_call(
            _routing_kernel,
            out_shape=[
                jax.ShapeDtypeStruct((bsz, _R, d_in), jnp.float32),
                jax.ShapeDtypeStruct((bsz, _R, d_out), jnp.float32),
            ],
            interpret=interpret,
        )(query_signal, W_A, b_A.reshape(1, n_exp), W_B,
          b_B.reshape(1, n_exp), A_pool, cfs_W, bt)

        s_blk = 512
        out = pl.pallas_call(
            _apply_kernel,
            grid=(bsz, seq // s_blk),
            in_specs=[
                pl.BlockSpec((1, s_blk, d_in), lambda b, s: (b, s, 0)),
                pl.BlockSpec((1, _R, d_in), lambda b, s: (b, 0, 0)),
                pl.BlockSpec((1, _R, d_out), lambda b, s: (b, 0, 0)),
            ],
            out_specs=pl.BlockSpec((1, s_blk, d_out), lambda b, s: (b, s, 0)),
            out_shape=jax.ShapeDtypeStruct((bsz, seq, d_out), jnp.float32),
            compiler_params=pltpu.CompilerParams(
                dimension_semantics=("parallel", "arbitrary")),
            interpret=interpret,
        )(x, lora_a, b_sel)
        return out
    return run


def kernel(x, query_signal, A_pool, B_pool, W_A, b_A, W_B, b_B, cfs_W):
    return _build(interpret=False)(x, query_signal, A_pool, B_pool,
                                   W_A, b_A, W_B, b_B, cfs_W)
